# baseline (speedup 1.0000x reference)
import functools

import jax
import jax.numpy as jnp
from jax import lax
from jax.experimental import pallas as pl
from jax.experimental.pallas import tpu as pltpu
from jax.experimental.pallas import tpu_sc as plsc

NC = 2
NS = 16
NW = NC * NS


def _make_gather(batch: int, hist: int, vocab: int, d: int):
    rows_per_w = batch // NW
    K = 12
    F = 8
    N = rows_per_w
    assert (N - K - F) % K == 0
    nround = (N - K - F) // K
    mesh = plsc.VectorSubcoreMesh(core_axis_name="c", subcore_axis_name="s")

    @functools.partial(
        pl.kernel,
        mesh=mesh,
        out_type=jax.ShapeDtypeStruct((batch, hist, d), jnp.float32),
        scratch_types=[
            pltpu.VMEM((rows_per_w, hist), jnp.int32),
        ]
        + [pltpu.VMEM((hist, d), jnp.float32) for _ in range(K)]
        + [pltpu.SemaphoreType.DMA for _ in range(K)],
    )
    def gather_kernel(table_hbm, idx_hbm, out_hbm, idx_v, *scratch):
        bufs = scratch[:K]
        sems = scratch[K:]
        wid = lax.axis_index("s") * NC + lax.axis_index("c")
        base = wid * rows_per_w
        pltpu.sync_copy(idx_hbm.at[pl.ds(base, rows_per_w)], idx_v)

        def gstart(i, s):
            pltpu.async_copy(table_hbm.at[idx_v.at[i]], bufs[s], sems[s])

        def gwait(i, s):
            pltpu.make_async_copy(table_hbm.at[idx_v.at[i]], bufs[s], sems[s]).wait()

        def sstart(i, s):
            pltpu.async_copy(bufs[s], out_hbm.at[base + i], sems[s])

        def swait(i, s):
            pltpu.make_async_copy(bufs[s], out_hbm.at[base + i], sems[s]).wait()


        for r in range(F):
            gstart(r, r % K)

        for i in range(K):
            gwait(i, i % K)
            sstart(i, i % K)
            j = i + F
            if j >= K:
                swait(j - K, j % K)
            gstart(j, j % K)

        def round_body(g, _):
            i0 = K + g * K
            for t in range(K):
                i = i0 + t
                gwait(i, t)
                sstart(i, t)
                sj = (t + F) % K
                swait(i + F - K, sj)
                gstart(i + F, sj)
            return _

        lax.fori_loop(0, nround, round_body, 0, unroll=False)

        for e in range(F):
            i = N - F + e
            gwait(i, i % K)
            sstart(i, i % K)

        for r in range(N - K, N):
            swait(r, r % K)

    return gather_kernel


def kernel(indices, table):
    b, h = indices.shape
    v, d = table.shape
    return _make_gather(b, h, v, d)(table, indices)
